# baseline (speedup 1.0000x reference)
import functools

import jax
import jax.numpy as jnp
from jax import lax
from jax.experimental import pallas as pl
from jax.experimental.pallas import tpu as pltpu
from jax.experimental.pallas import tpu_sc as plsc


def _sc_gather(table, idx):
    B = idx.shape[0]
    Hd2 = table.shape[1]
    info = plsc.get_sparse_core_info()
    nw = info.num_cores * info.num_subcores
    b_per_w = B // nw
    mesh = plsc.VectorSubcoreMesh(core_axis_name="c", subcore_axis_name="s")

    @functools.partial(
        pl.kernel,
        mesh=mesh,
        out_type=jax.ShapeDtypeStruct((B, Hd2), jnp.float32),
        scratch_types=[
            pltpu.VMEM((b_per_w,), jnp.int32),
            pltpu.VMEM((b_per_w, Hd2), jnp.float32),
            pltpu.SemaphoreType.DMA,
        ],
    )
    def gather_k(tab, ix, out, idx_v, rows_v, s1):
        wid = lax.axis_index("s") * info.num_cores + lax.axis_index("c")
        base = wid * b_per_w
        pltpu.sync_copy(ix.at[pl.ds(base, b_per_w)], idx_v)
        pltpu.async_copy(tab.at[idx_v], rows_v, s1).wait()
        pltpu.sync_copy(rows_v, out.at[pl.ds(base, b_per_w)])

    return gather_k(table, idx)


def _sort_body(parent_s, butree_s, cnt_td, ord_td, cnt_bu, ord_bu, maxs,
               dep_td, dep_bu):
    n = parent_s.shape[0]
    p = butree_s.shape[0]
    l = n - p
    d = butree_s.shape[1]


    _UF = 8
    dep_td[0] = 0

    def dtd(g, maxd):
        for q in range(_UF):
            i = g * _UF + q
            dv = dep_td[parent_s[i]] + 1
            dep_td[i + 1] = dv
            maxd = jnp.maximum(maxd, dv)
        return maxd

    maxd_td = lax.fori_loop(0, n // _UF, dtd, 0)

    def dbu0(g, c):
        for q in range(_UF):
            dep_bu[g * _UF + q] = 0
        return c

    lax.fori_loop(0, l // _UF, dbu0, 0)

    def dbu(g, maxd):
        for q in range(_UF):
            i = g * _UF + q
            dv = dep_bu[butree_s[i, 0]]
            for j in range(1, d):
                dv = jnp.maximum(dv, dep_bu[butree_s[i, j]])
            dv = dv + 1
            dep_bu[l + i] = dv
            maxd = jnp.maximum(maxd, dv)
        return maxd

    maxd_bu = lax.fori_loop(0, p // _UF, dbu, 0)

    maxdm = jnp.maximum(maxd_td, maxd_bu)

    def zcnt_td(k, c):
        cnt_td[k] = 0
        return c

    lax.fori_loop(0, maxdm + 2, zcnt_td, 0)

    def ctd(g, c):
        for q in range(_UF):
            dv = dep_td[g * _UF + q + 1]
            cnt_td[dv] = cnt_td[dv] + 1
        return c

    lax.fori_loop(0, n // _UF, ctd, 0)

    def ptd(k, run):
        cv = cnt_td[k]
        cnt_td[k] = run
        return run + cv

    lax.fori_loop(0, maxdm + 2, ptd, 0)

    def otd(g, c):
        for q in range(_UF):
            j = g * _UF + q + 1
            dv = dep_td[j]
            s = cnt_td[dv]
            ord_td[s] = j
            cnt_td[dv] = s + 1
        return c

    lax.fori_loop(0, n // _UF, otd, 0)

    def zcnt_bu(k, c):
        cnt_bu[k] = 0
        return c

    lax.fori_loop(0, maxdm + 2, zcnt_bu, 0)

    def cbu(g, c):
        for q in range(_UF):
            dv = dep_bu[l + g * _UF + q]
            cnt_bu[dv] = cnt_bu[dv] + 1
        return c

    lax.fori_loop(0, p // _UF, cbu, 0)

    def pbu(k, run):
        cv = cnt_bu[k]
        cnt_bu[k] = run
        return run + cv

    lax.fori_loop(0, maxdm + 2, pbu, 0)

    def obu(g, c):
        for q in range(_UF):
            i = g * _UF + q
            dv = dep_bu[l + i]
            s = cnt_bu[dv]
            ord_bu[s] = l + i
            cnt_bu[dv] = s + 1
        return c

    lax.fori_loop(0, p // _UF, obu, 0)

    maxs[0] = maxdm
    maxs[1] = maxd_td
    maxs[2] = maxd_bu


_TB = 64
_TBB = 32


def _tc_body(rows_f, xw, leafs, parent_s, butree_s,
             cnt_td, ord_td, cnt_bu, ord_bu, maxs,
             pzr_td, ph_td, bzr_td, bh_td,
             pzr_bu, ph_bu, bzr_bu, bh_bu,
             m1f, m2f,
             wo1, bo1, wo4, bo4,
             out_ref,
             h_td, h_bu, xe_s, xep_td, xep_bu,
             zrp_td, hpp_td, zrp_bu, hpp_bu,
             stg_h):
    n, w, hd2 = rows_f.shape
    hd = hd2 // 2
    p = butree_s.shape[0]
    l = n - p
    d = butree_s.shape[1]

    xww = xw[...][:, :, None]
    xe_s[...] = (rows_f[...] * xww).sum(axis=1)

    xe_bu_lo = xe_s[:l, hd:]
    z_l = jax.nn.sigmoid(jnp.dot(xe_bu_lo, pzr_bu[:, :hd],
                                 preferred_element_type=jnp.float32) +
                         bzr_bu[:, :hd])
    c_l = jnp.tanh(jnp.dot(xe_bu_lo, ph_bu[...],
                           preferred_element_type=jnp.float32) + bh_bu[...])
    h_bu[:l, :] = (1.0 - z_l) * c_l

    h_td[:1, :] = jnp.zeros((1, hd), jnp.float32)
    stg_h[...] = jnp.zeros((_TB, 2 * hd), jnp.float32)
    xep_td[n:, :] = jnp.zeros((_TB, hd), jnp.float32)
    xep_bu[p:, :] = jnp.zeros((_TBB, hd), jnp.float32)

    m1f_v = m1f[...]
    m2f_v = m2f[...]

    maxdm = maxs[0]

    _PF = 16

    def perm_td(g, c):
        for q in range(_PF):
            s = g * _PF + q
            j = ord_td[s]
            xep_td[pl.ds(s, 1), :] = xe_s[pl.ds(j - 1, 1), :hd]
        return c

    lax.fori_loop(0, n // _PF, perm_td, 0)

    def perm_bu(g, c):
        for q in range(_PF):
            s = g * _PF + q
            m = ord_bu[s]
            xep_bu[pl.ds(s, 1), :] = xe_s[pl.ds(m, 1), hd:]
        return c

    lax.fori_loop(0, p // _PF, perm_bu, 0)

    zrp_td[...] = jnp.dot(xep_td[...], pzr_td[...],
                          preferred_element_type=jnp.float32) + bzr_td[...]
    hpp_td[...] = jnp.dot(xep_td[...], ph_td[...],
                          preferred_element_type=jnp.float32) + bh_td[...]
    zrp_bu[...] = jnp.dot(xep_bu[...], pzr_bu[...],
                          preferred_element_type=jnp.float32) + bzr_bu[...]
    hpp_bu[...] = jnp.dot(xep_bu[...], ph_bu[...],
                          preferred_element_type=jnp.float32) + bh_bu[...]

    trash_td = n + 1
    trash_bu = n
    zpad_zr = jnp.zeros((_TB - _TBB, 2 * hd), jnp.float32)
    zpad_h = jnp.zeros((_TB - _TBB, hd), jnp.float32)

    def fused_tile(t_t, e_t, t_b, e_b):
        jst = []
        for s in range(_TB):
            pos = t_t + s
            pos_c = jnp.minimum(pos, e_t - 1)
            j = ord_td[pos_c]
            jst.append(jnp.where(pos < e_t, j, trash_td))
            pi = parent_s[j - 1]
            stg_h[pl.ds(s, 1), :hd] = h_td[pl.ds(pi, 1), :]
        mst = []
        for s in range(_TBB):
            pos = t_b + s
            pos_c = jnp.minimum(pos, e_b - 1)
            m = ord_bu[pos_c]
            mst.append(jnp.where(pos < e_b, m, trash_bu))
            i = m - l
            acc = h_bu[pl.ds(butree_s[i, 0], 1), :]
            for j in range(1, d):
                acc = acc + h_bu[pl.ds(butree_s[i, j], 1), :]
            stg_h[pl.ds(s, 1), hd:] = acc
        inp = stg_h[...]
        phb = inp[:, :hd]
        accb = inp[:_TBB, hd:]
        pre4 = jnp.concatenate(
            [zrp_td[pl.ds(t_t, _TB), :],
             jnp.concatenate([zrp_bu[pl.ds(t_b, _TBB), :], zpad_zr], axis=0)],
            axis=1)
        zr4 = jax.nn.sigmoid(pre4 +
                             jnp.dot(inp, m1f_v,
                                     preferred_element_type=jnp.float32))
        z_t = zr4[:, :hd]
        r_t = zr4[:, hd:2 * hd]
        z_b = zr4[:_TBB, 2 * hd:3 * hd]
        r_b = zr4[:_TBB, 3 * hd:]
        inp2 = jnp.concatenate(
            [phb * r_t,
             jnp.concatenate([accb * r_b, zpad_h], axis=0)], axis=1)
        hp2 = jnp.concatenate(
            [hpp_td[pl.ds(t_t, _TB), :],
             jnp.concatenate([hpp_bu[pl.ds(t_b, _TBB), :], zpad_h], axis=0)],
            axis=1)
        cc = jnp.tanh(hp2 + jnp.dot(inp2, m2f_v,
                                    preferred_element_type=jnp.float32))
        stg_h[:, :hd] = z_t * phb + (1.0 - z_t) * cc[:, :hd]
        stg_h[:_TBB, hd:] = z_b * accb + (1.0 - z_b) * cc[:_TBB, hd:]
        for s in range(_TB):
            h_td[pl.ds(jst[s], 1), :] = stg_h[pl.ds(s, 1), :hd]
        for s in range(_TBB):
            h_bu[pl.ds(mst[s], 1), :] = stg_h[pl.ds(s, 1), hd:]

    def level(k, c):
        b_t = cnt_td[k - 1]
        e_t = cnt_td[k]
        b_b = cnt_bu[k - 1]
        e_b = cnt_bu[k]

        def tile_cond(ts):
            return jnp.logical_or(ts[0] < e_t, ts[1] < e_b)

        def tile_body(ts):
            fused_tile(ts[0], e_t, ts[1], e_b)
            return (ts[0] + _TB, ts[1] + _TBB)

        lax.while_loop(tile_cond, tile_body, (b_t, b_b))
        return c

    lax.fori_loop(1, maxdm + 1, level, 0)

    row_ids = lax.broadcasted_iota(jnp.int32, (n + 1, leafs.shape[1]), 0)
    is_leaf = jnp.any(row_ids == leafs[...], axis=1, keepdims=True)
    td_final = jnp.max(jnp.where(is_leaf, h_td[:n + 1, :], -jnp.inf),
                       axis=0, keepdims=True)
    bu_root = h_bu[pl.ds(n - 1, 1), :]

    fs = jnp.concatenate([td_final, bu_root], axis=1)
    fs1 = jnp.maximum(jnp.dot(fs, wo1[...],
                              preferred_element_type=jnp.float32) + bo1[...],
                      0.0)
    logits = jnp.dot(fs1, wo4[...],
                     preferred_element_type=jnp.float32) + bo4[...]
    m = jnp.max(logits, axis=1, keepdims=True)
    e = jnp.exp(logits - m)
    out_ref[...] = e / jnp.sum(e, axis=1, keepdims=True)


def kernel(x_word, x_index, td_parent, bu_tree, leaf_idxs,
           E_td, W_z_td, U_z_td, b_z_td, W_r_td, U_r_td, b_r_td,
           W_h_td, U_h_td, b_h_td,
           E_bu, W_z_bu, U_z_bu, b_z_bu, W_r_bu, U_r_bu, b_r_bu,
           W_h_bu, U_h_bu, b_h_bu,
           W_out1, b_out1, W_out4, b_out4):
    n, w = x_word.shape
    p, d = bu_tree.shape
    hd = E_td.shape[0]
    nc = W_out4.shape[0]

    idx = x_index.astype(jnp.int32).reshape(n * w)
    table = jnp.concatenate([E_td, E_bu], axis=0).T
    rows_f = _sc_gather(table, idx).reshape(n, w, 2 * hd)

    pzr_td = jnp.concatenate([W_z_td.T, W_r_td.T], axis=1)
    bzr_td = jnp.concatenate([b_z_td, b_r_td])[None, :]
    uzr_td = jnp.concatenate([U_z_td.T, U_r_td.T], axis=1)
    pzr_bu = jnp.concatenate([W_z_bu.T, W_r_bu.T], axis=1)
    bzr_bu = jnp.concatenate([b_z_bu, b_r_bu])[None, :]
    uzr_bu = jnp.concatenate([U_z_bu.T, U_r_bu.T], axis=1)
    zz = jnp.zeros((hd, 2 * hd), jnp.float32)
    m1f = jnp.concatenate([jnp.concatenate([uzr_td, zz], axis=1),
                           jnp.concatenate([zz, uzr_bu], axis=1)], axis=0)
    zh = jnp.zeros((hd, hd), jnp.float32)
    m2f = jnp.concatenate([jnp.concatenate([U_h_td.T, zh], axis=1),
                           jnp.concatenate([zh, U_h_bu.T], axis=1)], axis=0)

    vmem = pl.BlockSpec(memory_space=pltpu.VMEM)
    smem = pl.BlockSpec(memory_space=pltpu.SMEM)

    sort_out = pl.pallas_call(
        _sort_body,
        out_shape=(
            jax.ShapeDtypeStruct((n + 2,), jnp.int32),
            jax.ShapeDtypeStruct((n,), jnp.int32),
            jax.ShapeDtypeStruct((p + 2,), jnp.int32),
            jax.ShapeDtypeStruct((p,), jnp.int32),
            jax.ShapeDtypeStruct((4,), jnp.int32),
        ),
        in_specs=[smem, smem],
        out_specs=(smem, smem, smem, smem, smem),
        scratch_shapes=[
            pltpu.SMEM((n + 1,), jnp.int32),
            pltpu.SMEM((n,), jnp.int32),
        ],
    )(td_parent.astype(jnp.int32), bu_tree.astype(jnp.int32))

    out = pl.pallas_call(
        _tc_body,
        out_shape=jax.ShapeDtypeStruct((1, nc), jnp.float32),
        in_specs=[vmem, vmem, vmem, smem, smem,
                  smem, smem, smem, smem, smem] + [vmem] * 14,
        out_specs=vmem,
        scratch_shapes=[
            pltpu.VMEM((n + 2, hd), jnp.float32),
            pltpu.VMEM((n + 1, hd), jnp.float32),
            pltpu.VMEM((n, 2 * hd), jnp.float32),
            pltpu.VMEM((n + _TB, hd), jnp.float32),
            pltpu.VMEM((p + _TBB, hd), jnp.float32),
            pltpu.VMEM((n + _TB, 2 * hd), jnp.float32),
            pltpu.VMEM((n + _TB, hd), jnp.float32),
            pltpu.VMEM((p + _TBB, 2 * hd), jnp.float32),
            pltpu.VMEM((p + _TBB, hd), jnp.float32),
            pltpu.VMEM((_TB, 2 * hd), jnp.float32),
        ],
    )(
        rows_f, x_word,
        leaf_idxs.astype(jnp.int32).reshape(1, -1),
        td_parent.astype(jnp.int32),
        bu_tree.astype(jnp.int32),
        sort_out[0], sort_out[1], sort_out[2], sort_out[3], sort_out[4],
        pzr_td, W_h_td.T, bzr_td, b_h_td[None, :],
        pzr_bu, W_h_bu.T, bzr_bu, b_h_bu[None, :],
        m1f, m2f,
        W_out1.T, b_out1[None, :], W_out4.T, b_out4[None, :],
    )
    return out[0]


# speedup vs baseline: 1.0289x; 1.0289x over previous
import functools

import jax
import jax.numpy as jnp
from jax import lax
from jax.experimental import pallas as pl
from jax.experimental.pallas import tpu as pltpu
from jax.experimental.pallas import tpu_sc as plsc


def _sc_gather(table, idx):
    B = idx.shape[0]
    Hd2 = table.shape[1]
    info = plsc.get_sparse_core_info()
    nw = info.num_cores * info.num_subcores
    b_per_w = B // nw
    mesh = plsc.VectorSubcoreMesh(core_axis_name="c", subcore_axis_name="s")

    @functools.partial(
        pl.kernel,
        mesh=mesh,
        out_type=jax.ShapeDtypeStruct((B, Hd2), jnp.float32),
        scratch_types=[
            pltpu.VMEM((b_per_w,), jnp.int32),
            pltpu.VMEM((b_per_w, Hd2), jnp.float32),
            pltpu.SemaphoreType.DMA,
        ],
    )
    def gather_k(tab, ix, out, idx_v, rows_v, s1):
        wid = lax.axis_index("s") * info.num_cores + lax.axis_index("c")
        base = wid * b_per_w
        pltpu.sync_copy(ix.at[pl.ds(base, b_per_w)], idx_v)
        pltpu.async_copy(tab.at[idx_v], rows_v, s1).wait()
        pltpu.sync_copy(rows_v, out.at[pl.ds(base, b_per_w)])

    return gather_k(table, idx)


_TB = 48
_TBB = 32


def _tc_body(rows_f, xw, leafs, parent_s, butree_s,
             pzr_td, ph_td, bzr_td, bh_td,
             pzr_bu, ph_bu, bzr_bu, bh_bu,
             m1f, m2f,
             wo1, bo1, wo4, bo4,
             out_ref,
             h_td, h_bu, xe_s, xep_td, xep_bu,
             zrp_td, hpp_td, zrp_bu, hpp_bu,
             stg_h,
             dep_td, cnt_td, ord_td, dep_bu, cnt_bu, ord_bu):
    n, w, hd2 = rows_f.shape
    hd = hd2 // 2
    p = butree_s.shape[0]
    l = n - p
    d = butree_s.shape[1]

    xww = xw[...][:, :, None]
    xe_s[...] = (rows_f[...] * xww).sum(axis=1)

    xe_bu_lo = xe_s[:l, hd:]
    z_l = jax.nn.sigmoid(jnp.dot(xe_bu_lo, pzr_bu[:, :hd],
                                 preferred_element_type=jnp.float32) +
                         bzr_bu[:, :hd])
    c_l = jnp.tanh(jnp.dot(xe_bu_lo, ph_bu[...],
                           preferred_element_type=jnp.float32) + bh_bu[...])
    h_bu[:l, :] = (1.0 - z_l) * c_l

    h_td[:1, :] = jnp.zeros((1, hd), jnp.float32)
    stg_h[...] = jnp.zeros((_TB, 2 * hd), jnp.float32)
    xep_td[n:, :] = jnp.zeros((_TB, hd), jnp.float32)
    xep_bu[p:, :] = jnp.zeros((_TBB, hd), jnp.float32)

    m1f_v = m1f[...]
    m2f_v = m2f[...]


    _UF = 8
    dep_td[0] = 0

    def dtd(g, maxd):
        for q in range(_UF):
            i = g * _UF + q
            dv = dep_td[parent_s[i]] + 1
            dep_td[i + 1] = dv
            maxd = jnp.maximum(maxd, dv)
        return maxd

    maxd_td = lax.fori_loop(0, n // _UF, dtd, 0)

    def dbu0(g, c):
        for q in range(_UF):
            dep_bu[g * _UF + q] = 0
        return c

    lax.fori_loop(0, l // _UF, dbu0, 0)

    def dbu(g, maxd):
        for q in range(_UF):
            i = g * _UF + q
            dv = dep_bu[butree_s[i, 0]]
            for j in range(1, d):
                dv = jnp.maximum(dv, dep_bu[butree_s[i, j]])
            dv = dv + 1
            dep_bu[l + i] = dv
            maxd = jnp.maximum(maxd, dv)
        return maxd

    maxd_bu = lax.fori_loop(0, p // _UF, dbu, 0)

    maxdm = jnp.maximum(maxd_td, maxd_bu)

    def zcnt_td(k, c):
        cnt_td[k] = 0
        return c

    lax.fori_loop(0, maxdm + 2, zcnt_td, 0)

    def ctd(g, c):
        for q in range(_UF):
            dv = dep_td[g * _UF + q + 1]
            cnt_td[dv] = cnt_td[dv] + 1
        return c

    lax.fori_loop(0, n // _UF, ctd, 0)

    def ptd(k, run):
        cv = cnt_td[k]
        cnt_td[k] = run
        return run + cv

    lax.fori_loop(0, maxdm + 2, ptd, 0)

    def otd(g, c):
        for q in range(_UF):
            j = g * _UF + q + 1
            dv = dep_td[j]
            s = cnt_td[dv]
            ord_td[s] = j
            cnt_td[dv] = s + 1
        return c

    lax.fori_loop(0, n // _UF, otd, 0)

    def zcnt_bu(k, c):
        cnt_bu[k] = 0
        return c

    lax.fori_loop(0, maxdm + 2, zcnt_bu, 0)

    def cbu(g, c):
        for q in range(_UF):
            dv = dep_bu[l + g * _UF + q]
            cnt_bu[dv] = cnt_bu[dv] + 1
        return c

    lax.fori_loop(0, p // _UF, cbu, 0)

    def pbu(k, run):
        cv = cnt_bu[k]
        cnt_bu[k] = run
        return run + cv

    lax.fori_loop(0, maxdm + 2, pbu, 0)

    def obu(g, c):
        for q in range(_UF):
            i = g * _UF + q
            dv = dep_bu[l + i]
            s = cnt_bu[dv]
            ord_bu[s] = l + i
            cnt_bu[dv] = s + 1
        return c

    lax.fori_loop(0, p // _UF, obu, 0)

    _PF = 16

    def perm_td(g, c):
        for q in range(_PF):
            s = g * _PF + q
            j = ord_td[s]
            xep_td[pl.ds(s, 1), :] = xe_s[pl.ds(j - 1, 1), :hd]
        return c

    lax.fori_loop(0, n // _PF, perm_td, 0)

    def perm_bu(g, c):
        for q in range(_PF):
            s = g * _PF + q
            m = ord_bu[s]
            xep_bu[pl.ds(s, 1), :] = xe_s[pl.ds(m, 1), hd:]
        return c

    lax.fori_loop(0, p // _PF, perm_bu, 0)

    zrp_td[...] = jnp.dot(xep_td[...], pzr_td[...],
                          preferred_element_type=jnp.float32) + bzr_td[...]
    hpp_td[...] = jnp.dot(xep_td[...], ph_td[...],
                          preferred_element_type=jnp.float32) + bh_td[...]
    zrp_bu[...] = jnp.dot(xep_bu[...], pzr_bu[...],
                          preferred_element_type=jnp.float32) + bzr_bu[...]
    hpp_bu[...] = jnp.dot(xep_bu[...], ph_bu[...],
                          preferred_element_type=jnp.float32) + bh_bu[...]

    trash_td = n + 1
    trash_bu = n
    zpad_zr = jnp.zeros((_TB - _TBB, 2 * hd), jnp.float32)
    zpad_h = jnp.zeros((_TB - _TBB, hd), jnp.float32)

    def fused_tile(t_t, e_t, t_b, e_b):
        jst = []
        for s in range(_TB):
            pos = t_t + s
            pos_c = jnp.minimum(pos, e_t - 1)
            j = ord_td[pos_c]
            jst.append(jnp.where(pos < e_t, j, trash_td))
            pi = parent_s[j - 1]
            stg_h[pl.ds(s, 1), :hd] = h_td[pl.ds(pi, 1), :]
        mst = []
        for s in range(_TBB):
            pos = t_b + s
            pos_c = jnp.minimum(pos, e_b - 1)
            m = ord_bu[pos_c]
            mst.append(jnp.where(pos < e_b, m, trash_bu))
            i = m - l
            acc = h_bu[pl.ds(butree_s[i, 0], 1), :]
            for j in range(1, d):
                acc = acc + h_bu[pl.ds(butree_s[i, j], 1), :]
            stg_h[pl.ds(s, 1), hd:] = acc
        inp = stg_h[...]
        phb = inp[:, :hd]
        accb = inp[:_TBB, hd:]
        pre4 = jnp.concatenate(
            [zrp_td[pl.ds(t_t, _TB), :],
             jnp.concatenate([zrp_bu[pl.ds(t_b, _TBB), :], zpad_zr], axis=0)],
            axis=1)
        zr4 = jax.nn.sigmoid(pre4 +
                             jnp.dot(inp, m1f_v,
                                     preferred_element_type=jnp.float32))
        z_t = zr4[:, :hd]
        r_t = zr4[:, hd:2 * hd]
        z_b = zr4[:_TBB, 2 * hd:3 * hd]
        r_b = zr4[:_TBB, 3 * hd:]
        inp2 = jnp.concatenate(
            [phb * r_t,
             jnp.concatenate([accb * r_b, zpad_h], axis=0)], axis=1)
        hp2 = jnp.concatenate(
            [hpp_td[pl.ds(t_t, _TB), :],
             jnp.concatenate([hpp_bu[pl.ds(t_b, _TBB), :], zpad_h], axis=0)],
            axis=1)
        cc = jnp.tanh(hp2 + jnp.dot(inp2, m2f_v,
                                    preferred_element_type=jnp.float32))
        stg_h[:, :hd] = z_t * phb + (1.0 - z_t) * cc[:, :hd]
        stg_h[:_TBB, hd:] = z_b * accb + (1.0 - z_b) * cc[:_TBB, hd:]
        for s in range(_TB):
            h_td[pl.ds(jst[s], 1), :] = stg_h[pl.ds(s, 1), :hd]
        for s in range(_TBB):
            h_bu[pl.ds(mst[s], 1), :] = stg_h[pl.ds(s, 1), hd:]

    def level(k, c):
        b_t = cnt_td[k - 1]
        e_t = cnt_td[k]
        b_b = cnt_bu[k - 1]
        e_b = cnt_bu[k]

        def tile_cond(ts):
            return jnp.logical_or(ts[0] < e_t, ts[1] < e_b)

        def tile_body(ts):
            fused_tile(ts[0], e_t, ts[1], e_b)
            return (ts[0] + _TB, ts[1] + _TBB)

        lax.while_loop(tile_cond, tile_body, (b_t, b_b))
        return c

    lax.fori_loop(1, maxdm + 1, level, 0)

    row_ids = lax.broadcasted_iota(jnp.int32, (n + 1, leafs.shape[1]), 0)
    is_leaf = jnp.any(row_ids == leafs[...], axis=1, keepdims=True)
    td_final = jnp.max(jnp.where(is_leaf, h_td[:n + 1, :], -jnp.inf),
                       axis=0, keepdims=True)
    bu_root = h_bu[pl.ds(n - 1, 1), :]

    fs = jnp.concatenate([td_final, bu_root], axis=1)
    fs1 = jnp.maximum(jnp.dot(fs, wo1[...],
                              preferred_element_type=jnp.float32) + bo1[...],
                      0.0)
    logits = jnp.dot(fs1, wo4[...],
                     preferred_element_type=jnp.float32) + bo4[...]
    m = jnp.max(logits, axis=1, keepdims=True)
    e = jnp.exp(logits - m)
    out_ref[...] = e / jnp.sum(e, axis=1, keepdims=True)


def kernel(x_word, x_index, td_parent, bu_tree, leaf_idxs,
           E_td, W_z_td, U_z_td, b_z_td, W_r_td, U_r_td, b_r_td,
           W_h_td, U_h_td, b_h_td,
           E_bu, W_z_bu, U_z_bu, b_z_bu, W_r_bu, U_r_bu, b_r_bu,
           W_h_bu, U_h_bu, b_h_bu,
           W_out1, b_out1, W_out4, b_out4):
    n, w = x_word.shape
    p, d = bu_tree.shape
    hd = E_td.shape[0]
    nc = W_out4.shape[0]

    idx = x_index.astype(jnp.int32).reshape(n * w)
    table = jnp.concatenate([E_td, E_bu], axis=0).T
    rows_f = _sc_gather(table, idx).reshape(n, w, 2 * hd)

    pzr_td = jnp.concatenate([W_z_td.T, W_r_td.T], axis=1)
    bzr_td = jnp.concatenate([b_z_td, b_r_td])[None, :]
    uzr_td = jnp.concatenate([U_z_td.T, U_r_td.T], axis=1)
    pzr_bu = jnp.concatenate([W_z_bu.T, W_r_bu.T], axis=1)
    bzr_bu = jnp.concatenate([b_z_bu, b_r_bu])[None, :]
    uzr_bu = jnp.concatenate([U_z_bu.T, U_r_bu.T], axis=1)
    zz = jnp.zeros((hd, 2 * hd), jnp.float32)
    m1f = jnp.concatenate([jnp.concatenate([uzr_td, zz], axis=1),
                           jnp.concatenate([zz, uzr_bu], axis=1)], axis=0)
    zh = jnp.zeros((hd, hd), jnp.float32)
    m2f = jnp.concatenate([jnp.concatenate([U_h_td.T, zh], axis=1),
                           jnp.concatenate([zh, U_h_bu.T], axis=1)], axis=0)

    vmem = pl.BlockSpec(memory_space=pltpu.VMEM)
    smem = pl.BlockSpec(memory_space=pltpu.SMEM)

    out = pl.pallas_call(
        _tc_body,
        out_shape=jax.ShapeDtypeStruct((1, nc), jnp.float32),
        in_specs=[vmem, vmem, vmem, smem, smem] + [vmem] * 14,
        out_specs=vmem,
        scratch_shapes=[
            pltpu.VMEM((n + 2, hd), jnp.float32),
            pltpu.VMEM((n + 1, hd), jnp.float32),
            pltpu.VMEM((n, 2 * hd), jnp.float32),
            pltpu.VMEM((n + _TB, hd), jnp.float32),
            pltpu.VMEM((p + _TBB, hd), jnp.float32),
            pltpu.VMEM((n + _TB, 2 * hd), jnp.float32),
            pltpu.VMEM((n + _TB, hd), jnp.float32),
            pltpu.VMEM((p + _TBB, 2 * hd), jnp.float32),
            pltpu.VMEM((p + _TBB, hd), jnp.float32),
            pltpu.VMEM((_TB, 2 * hd), jnp.float32),
            pltpu.SMEM((n + 1,), jnp.int32),
            pltpu.SMEM((n + 2,), jnp.int32),
            pltpu.SMEM((n,), jnp.int32),
            pltpu.SMEM((n,), jnp.int32),
            pltpu.SMEM((p + 2,), jnp.int32),
            pltpu.SMEM((p,), jnp.int32),
        ],
    )(
        rows_f, x_word,
        leaf_idxs.astype(jnp.int32).reshape(1, -1),
        td_parent.astype(jnp.int32),
        bu_tree.astype(jnp.int32),
        pzr_td, W_h_td.T, bzr_td, b_h_td[None, :],
        pzr_bu, W_h_bu.T, bzr_bu, b_h_bu[None, :],
        m1f, m2f,
        W_out1.T, b_out1[None, :], W_out4.T, b_out4[None, :],
    )
    return out[0]


# speedup vs baseline: 1.0609x; 1.0311x over previous
import functools

import jax
import jax.numpy as jnp
from jax import lax
from jax.experimental import pallas as pl
from jax.experimental.pallas import tpu as pltpu
from jax.experimental.pallas import tpu_sc as plsc


def _sc_gather(table, idx):
    B = idx.shape[0]
    Hd2 = table.shape[1]
    info = plsc.get_sparse_core_info()
    nw = info.num_cores * info.num_subcores
    b_per_w = B // nw
    mesh = plsc.VectorSubcoreMesh(core_axis_name="c", subcore_axis_name="s")

    @functools.partial(
        pl.kernel,
        mesh=mesh,
        out_type=jax.ShapeDtypeStruct((B, Hd2), jnp.float32),
        scratch_types=[
            pltpu.VMEM((b_per_w,), jnp.int32),
            pltpu.VMEM((b_per_w, Hd2), jnp.float32),
            pltpu.SemaphoreType.DMA,
        ],
    )
    def gather_k(tab, ix, out, idx_v, rows_v, s1):
        wid = lax.axis_index("s") * info.num_cores + lax.axis_index("c")
        base = wid * b_per_w
        pltpu.sync_copy(ix.at[pl.ds(base, b_per_w)], idx_v)
        pltpu.async_copy(tab.at[idx_v], rows_v, s1).wait()
        pltpu.sync_copy(rows_v, out.at[pl.ds(base, b_per_w)])

    return gather_k(table, idx)


_TB = 48
_TBB = 16


def _tc_body(rows_f, xw, leafs, parent_s, butree_s,
             pzr_td, ph_td, bzr_td, bh_td,
             pzr_bu, ph_bu, bzr_bu, bh_bu,
             m1f, m2f,
             wo1, bo1, wo4, bo4,
             out_ref,
             h_td, h_bu, xe_s, xep_td, xep_bu,
             zrp_td, hpp_td, zrp_bu, hpp_bu,
             stg_h,
             dep_td, cnt_td, ord_td, dep_bu, cnt_bu, ord_bu):
    n, w, hd2 = rows_f.shape
    hd = hd2 // 2
    p = butree_s.shape[0]
    l = n - p
    d = butree_s.shape[1]

    xww = xw[...][:, :, None]
    xe_s[...] = (rows_f[...] * xww).sum(axis=1)

    xe_bu_lo = xe_s[:l, hd:]
    z_l = jax.nn.sigmoid(jnp.dot(xe_bu_lo, pzr_bu[:, :hd],
                                 preferred_element_type=jnp.float32) +
                         bzr_bu[:, :hd])
    c_l = jnp.tanh(jnp.dot(xe_bu_lo, ph_bu[...],
                           preferred_element_type=jnp.float32) + bh_bu[...])
    h_bu[:l, :] = (1.0 - z_l) * c_l

    h_td[:1, :] = jnp.zeros((1, hd), jnp.float32)
    stg_h[...] = jnp.zeros((_TB, 2 * hd), jnp.float32)
    xep_td[n:, :] = jnp.zeros((_TB, hd), jnp.float32)
    xep_bu[p:, :] = jnp.zeros((_TBB, hd), jnp.float32)

    m1f_v = m1f[...]
    m2f_v = m2f[...]


    _UF = 8
    dep_td[0] = 0

    def dtd(g, maxd):
        for q in range(_UF):
            i = g * _UF + q
            dv = dep_td[parent_s[i]] + 1
            dep_td[i + 1] = dv
            maxd = jnp.maximum(maxd, dv)
        return maxd

    maxd_td = lax.fori_loop(0, n // _UF, dtd, 0)

    def dbu0(g, c):
        for q in range(_UF):
            dep_bu[g * _UF + q] = 0
        return c

    lax.fori_loop(0, l // _UF, dbu0, 0)

    def dbu(g, maxd):
        for q in range(_UF):
            i = g * _UF + q
            dv = dep_bu[butree_s[i, 0]]
            for j in range(1, d):
                dv = jnp.maximum(dv, dep_bu[butree_s[i, j]])
            dv = dv + 1
            dep_bu[l + i] = dv
            maxd = jnp.maximum(maxd, dv)
        return maxd

    maxd_bu = lax.fori_loop(0, p // _UF, dbu, 0)

    maxdm = jnp.maximum(maxd_td, maxd_bu)

    def zcnt_td(k, c):
        cnt_td[k] = 0
        return c

    lax.fori_loop(0, maxdm + 2, zcnt_td, 0)

    def ctd(g, c):
        for q in range(_UF):
            dv = dep_td[g * _UF + q + 1]
            cnt_td[dv] = cnt_td[dv] + 1
        return c

    lax.fori_loop(0, n // _UF, ctd, 0)

    def ptd(k, run):
        cv = cnt_td[k]
        cnt_td[k] = run
        return run + cv

    lax.fori_loop(0, maxdm + 2, ptd, 0)

    def otd(g, c):
        for q in range(_UF):
            j = g * _UF + q + 1
            dv = dep_td[j]
            s = cnt_td[dv]
            ord_td[s] = j
            cnt_td[dv] = s + 1
        return c

    lax.fori_loop(0, n // _UF, otd, 0)

    def zcnt_bu(k, c):
        cnt_bu[k] = 0
        return c

    lax.fori_loop(0, maxdm + 2, zcnt_bu, 0)

    def cbu(g, c):
        for q in range(_UF):
            dv = dep_bu[l + g * _UF + q]
            cnt_bu[dv] = cnt_bu[dv] + 1
        return c

    lax.fori_loop(0, p // _UF, cbu, 0)

    def pbu(k, run):
        cv = cnt_bu[k]
        cnt_bu[k] = run
        return run + cv

    lax.fori_loop(0, maxdm + 2, pbu, 0)

    def obu(g, c):
        for q in range(_UF):
            i = g * _UF + q
            dv = dep_bu[l + i]
            s = cnt_bu[dv]
            ord_bu[s] = l + i
            cnt_bu[dv] = s + 1
        return c

    lax.fori_loop(0, p // _UF, obu, 0)

    _PF = 16

    def perm_td(g, c):
        for q in range(_PF):
            s = g * _PF + q
            j = ord_td[s]
            xep_td[pl.ds(s, 1), :] = xe_s[pl.ds(j - 1, 1), :hd]
        return c

    lax.fori_loop(0, n // _PF, perm_td, 0)

    def perm_bu(g, c):
        for q in range(_PF):
            s = g * _PF + q
            m = ord_bu[s]
            xep_bu[pl.ds(s, 1), :] = xe_s[pl.ds(m, 1), hd:]
        return c

    lax.fori_loop(0, p // _PF, perm_bu, 0)

    zrp_td[...] = jnp.dot(xep_td[...], pzr_td[...],
                          preferred_element_type=jnp.float32) + bzr_td[...]
    hpp_td[...] = jnp.dot(xep_td[...], ph_td[...],
                          preferred_element_type=jnp.float32) + bh_td[...]
    zrp_bu[...] = jnp.dot(xep_bu[...], pzr_bu[...],
                          preferred_element_type=jnp.float32) + bzr_bu[...]
    hpp_bu[...] = jnp.dot(xep_bu[...], ph_bu[...],
                          preferred_element_type=jnp.float32) + bh_bu[...]

    trash_td = n + 1
    trash_bu = n
    zpad_zr = jnp.zeros((_TB - _TBB, 2 * hd), jnp.float32)
    zpad_h = jnp.zeros((_TB - _TBB, hd), jnp.float32)

    def fused_tile(t_t, e_t, t_b, e_b):
        jst = []
        for s in range(_TB):
            pos = t_t + s
            pos_c = jnp.minimum(pos, e_t - 1)
            j = ord_td[pos_c]
            jst.append(jnp.where(pos < e_t, j, trash_td))
            pi = parent_s[j - 1]
            stg_h[pl.ds(s, 1), :hd] = h_td[pl.ds(pi, 1), :]
        mst = []
        for s in range(_TBB):
            pos = t_b + s
            pos_c = jnp.minimum(pos, e_b - 1)
            m = ord_bu[pos_c]
            mst.append(jnp.where(pos < e_b, m, trash_bu))
            i = m - l
            acc = h_bu[pl.ds(butree_s[i, 0], 1), :]
            for j in range(1, d):
                acc = acc + h_bu[pl.ds(butree_s[i, j], 1), :]
            stg_h[pl.ds(s, 1), hd:] = acc
        inp = stg_h[...]
        phb = inp[:, :hd]
        accb = inp[:_TBB, hd:]
        pre4 = jnp.concatenate(
            [zrp_td[pl.ds(t_t, _TB), :],
             jnp.concatenate([zrp_bu[pl.ds(t_b, _TBB), :], zpad_zr], axis=0)],
            axis=1)
        zr4 = jax.nn.sigmoid(pre4 +
                             jnp.dot(inp, m1f_v,
                                     preferred_element_type=jnp.float32))
        z_t = zr4[:, :hd]
        r_t = zr4[:, hd:2 * hd]
        z_b = zr4[:_TBB, 2 * hd:3 * hd]
        r_b = zr4[:_TBB, 3 * hd:]
        inp2 = jnp.concatenate(
            [phb * r_t,
             jnp.concatenate([accb * r_b, zpad_h], axis=0)], axis=1)
        hp2 = jnp.concatenate(
            [hpp_td[pl.ds(t_t, _TB), :],
             jnp.concatenate([hpp_bu[pl.ds(t_b, _TBB), :], zpad_h], axis=0)],
            axis=1)
        cc = jnp.tanh(hp2 + jnp.dot(inp2, m2f_v,
                                    preferred_element_type=jnp.float32))
        stg_h[:, :hd] = z_t * phb + (1.0 - z_t) * cc[:, :hd]
        stg_h[:_TBB, hd:] = z_b * accb + (1.0 - z_b) * cc[:_TBB, hd:]
        for s in range(_TB):
            h_td[pl.ds(jst[s], 1), :] = stg_h[pl.ds(s, 1), :hd]
        for s in range(_TBB):
            h_bu[pl.ds(mst[s], 1), :] = stg_h[pl.ds(s, 1), hd:]

    def level(k, c):
        b_t = cnt_td[k - 1]
        e_t = cnt_td[k]
        b_b = cnt_bu[k - 1]
        e_b = cnt_bu[k]

        def tile_cond(ts):
            return jnp.logical_or(ts[0] < e_t, ts[1] < e_b)

        def tile_body(ts):
            fused_tile(ts[0], e_t, ts[1], e_b)
            return (ts[0] + _TB, ts[1] + _TBB)

        lax.while_loop(tile_cond, tile_body, (b_t, b_b))
        return c

    lax.fori_loop(1, maxdm + 1, level, 0)

    row_ids = lax.broadcasted_iota(jnp.int32, (n + 1, leafs.shape[1]), 0)
    is_leaf = jnp.any(row_ids == leafs[...], axis=1, keepdims=True)
    td_final = jnp.max(jnp.where(is_leaf, h_td[:n + 1, :], -jnp.inf),
                       axis=0, keepdims=True)
    bu_root = h_bu[pl.ds(n - 1, 1), :]

    fs = jnp.concatenate([td_final, bu_root], axis=1)
    fs1 = jnp.maximum(jnp.dot(fs, wo1[...],
                              preferred_element_type=jnp.float32) + bo1[...],
                      0.0)
    logits = jnp.dot(fs1, wo4[...],
                     preferred_element_type=jnp.float32) + bo4[...]
    m = jnp.max(logits, axis=1, keepdims=True)
    e = jnp.exp(logits - m)
    out_ref[...] = e / jnp.sum(e, axis=1, keepdims=True)


def kernel(x_word, x_index, td_parent, bu_tree, leaf_idxs,
           E_td, W_z_td, U_z_td, b_z_td, W_r_td, U_r_td, b_r_td,
           W_h_td, U_h_td, b_h_td,
           E_bu, W_z_bu, U_z_bu, b_z_bu, W_r_bu, U_r_bu, b_r_bu,
           W_h_bu, U_h_bu, b_h_bu,
           W_out1, b_out1, W_out4, b_out4):
    n, w = x_word.shape
    p, d = bu_tree.shape
    hd = E_td.shape[0]
    nc = W_out4.shape[0]

    idx = x_index.astype(jnp.int32).reshape(n * w)
    table = jnp.concatenate([E_td, E_bu], axis=0).T
    rows_f = _sc_gather(table, idx).reshape(n, w, 2 * hd)

    pzr_td = jnp.concatenate([W_z_td.T, W_r_td.T], axis=1)
    bzr_td = jnp.concatenate([b_z_td, b_r_td])[None, :]
    uzr_td = jnp.concatenate([U_z_td.T, U_r_td.T], axis=1)
    pzr_bu = jnp.concatenate([W_z_bu.T, W_r_bu.T], axis=1)
    bzr_bu = jnp.concatenate([b_z_bu, b_r_bu])[None, :]
    uzr_bu = jnp.concatenate([U_z_bu.T, U_r_bu.T], axis=1)
    zz = jnp.zeros((hd, 2 * hd), jnp.float32)
    m1f = jnp.concatenate([jnp.concatenate([uzr_td, zz], axis=1),
                           jnp.concatenate([zz, uzr_bu], axis=1)], axis=0)
    zh = jnp.zeros((hd, hd), jnp.float32)
    m2f = jnp.concatenate([jnp.concatenate([U_h_td.T, zh], axis=1),
                           jnp.concatenate([zh, U_h_bu.T], axis=1)], axis=0)

    vmem = pl.BlockSpec(memory_space=pltpu.VMEM)
    smem = pl.BlockSpec(memory_space=pltpu.SMEM)

    out = pl.pallas_call(
        _tc_body,
        out_shape=jax.ShapeDtypeStruct((1, nc), jnp.float32),
        in_specs=[vmem, vmem, vmem, smem, smem] + [vmem] * 14,
        out_specs=vmem,
        scratch_shapes=[
            pltpu.VMEM((n + 2, hd), jnp.float32),
            pltpu.VMEM((n + 1, hd), jnp.float32),
            pltpu.VMEM((n, 2 * hd), jnp.float32),
            pltpu.VMEM((n + _TB, hd), jnp.float32),
            pltpu.VMEM((p + _TBB, hd), jnp.float32),
            pltpu.VMEM((n + _TB, 2 * hd), jnp.float32),
            pltpu.VMEM((n + _TB, hd), jnp.float32),
            pltpu.VMEM((p + _TBB, 2 * hd), jnp.float32),
            pltpu.VMEM((p + _TBB, hd), jnp.float32),
            pltpu.VMEM((_TB, 2 * hd), jnp.float32),
            pltpu.SMEM((n + 1,), jnp.int32),
            pltpu.SMEM((n + 2,), jnp.int32),
            pltpu.SMEM((n,), jnp.int32),
            pltpu.SMEM((n,), jnp.int32),
            pltpu.SMEM((p + 2,), jnp.int32),
            pltpu.SMEM((p,), jnp.int32),
        ],
    )(
        rows_f, x_word,
        leaf_idxs.astype(jnp.int32).reshape(1, -1),
        td_parent.astype(jnp.int32),
        bu_tree.astype(jnp.int32),
        pzr_td, W_h_td.T, bzr_td, b_h_td[None, :],
        pzr_bu, W_h_bu.T, bzr_bu, b_h_bu[None, :],
        m1f, m2f,
        W_out1.T, b_out1[None, :], W_out4.T, b_out4[None, :],
    )
    return out[0]
